# baseline (speedup 1.0000x reference)
import functools

import jax
import jax.numpy as jnp
from jax import lax
from jax.experimental import pallas as pl
from jax.experimental.pallas import tpu as pltpu
from jax.experimental.pallas import tpu_sc as plsc

N = 10000
E = 320000
D = 128
NC = 2
NS = 16
NW = NC * NS
EPW = E // NW
K = 80
NCH = EPW // K
GRP = 5
NG = NCH // GRP
NP = 10240
RPT = NP // NS


def _phase0(x, Wm, A8):
    bn = 1000

    def body(x_ref, w_ref, a_ref, h_ref, s_ref):
        xb = x_ref[...]
        h_ref[...] = jnp.dot(xb, w_ref[...], preferred_element_type=jnp.float32)
        m = lax.dot_general(a_ref[...], xb, (((1,), (1,)), ((), ())),
                            preferred_element_type=jnp.float32)
        s_ref[...] = m[None]

    return pl.pallas_call(
        body,
        grid=(N // bn,),
        in_specs=[pl.BlockSpec((bn, D), lambda i: (i, 0)),
                  pl.BlockSpec((D, D), lambda i: (0, 0)),
                  pl.BlockSpec((8, D), lambda i: (0, 0))],
        out_specs=[pl.BlockSpec((bn, D), lambda i: (i, 0)),
                   pl.BlockSpec((1, 8, bn), lambda i: (i, 0, 0))],
        out_shape=[jax.ShapeDtypeStruct((N, D), jnp.float32),
                   jax.ShapeDtypeStruct((N // bn, 8, bn), jnp.float32)],
    )(x, Wm, A8)


def _make_edge_kernel():
    mesh = plsc.VectorSubcoreMesh(core_axis_name="c", subcore_axis_name="s")

    @functools.partial(
        pl.kernel,
        mesh=mesh,
        compiler_params=pltpu.CompilerParams(needs_layout_passes=False,
                                             use_tc_tiling_on_sc=False),
        out_type=[jax.ShapeDtypeStruct((NC, NP, D // 2), jnp.float32),
                  jax.ShapeDtypeStruct((NC, NP, D // 2), jnp.float32),
                  jax.ShapeDtypeStruct((NW, 1, N), jnp.float32)],
        scratch_types=[
            pltpu.VMEM((GRP, 1, K), jnp.int32),
            pltpu.VMEM((GRP, 1, K), jnp.int32),
            pltpu.VMEM((K,), jnp.int32),
            pltpu.VMEM((K,), jnp.int32),
            pltpu.VMEM((N,), jnp.float32),
            pltpu.VMEM((N,), jnp.float32),
            pltpu.VMEM((K,), jnp.float32),
            pltpu.VMEM((K, D // 2), jnp.float32),
            pltpu.VMEM((K, D // 2), jnp.float32),
            pltpu.VMEM((1, N), jnp.float32),
            pltpu.VMEM_SHARED((NP, D // 2), jnp.float32),
            pltpu.VMEM_SHARED((NP, D // 2), jnp.float32),
            pltpu.SemaphoreType.DMA,
            pltpu.SemaphoreType.DMA,
        ])
    def edge_kernel(src_hbm, dst_hbm, tsrc_hbm, tdst_hbm, h1_hbm, h2_hbm,
                    numer1_hbm, numer2_hbm, denom_hbm,
                    src_g, dst_g, src_c, dst_c, tsrc, tdst, wbuf,
                    rows_a, rows_b, dvec, numer1_s, numer2_s, sem_a, sem_b):
        cid = lax.axis_index("c")
        sid = lax.axis_index("s")
        wid = sid * NC + cid
        base = sid * RPT

        pltpu.sync_copy(tsrc_hbm, tsrc)
        pltpu.sync_copy(tdst_hbm, tdst)

        zv = jnp.zeros((16,), jnp.float32)

        def zdv(i, _):
            dvec[0, pl.ds(i * 16, 16)] = zv
            return 0

        lax.fori_loop(0, N // 16, zdv, 0)

        def zrow(r, _):
            for cc in range(D // 32):
                rows_a[r, pl.ds(cc * 16, 16)] = zv
                rows_b[r, pl.ds(cc * 16, 16)] = zv
            return 0

        lax.fori_loop(0, K, zrow, 0)

        def zcopy(j, _):
            pltpu.sync_copy(rows_a, numer1_s.at[pl.ds(base + j * K, K)])
            pltpu.sync_copy(rows_b, numer2_s.at[pl.ds(base + j * K, K)])
            return 0

        lax.fori_loop(0, RPT // K, zcopy, 0)
        plsc.subcore_barrier()

        zi = jnp.zeros((16,), jnp.int32)

        def group(g, _):
            pltpu.sync_copy(src_hbm.at[wid, g], src_g)
            pltpu.sync_copy(dst_hbm.at[wid, g], dst_g)

            def chunk(j, _):
                for v in range(K // 16):
                    src_c[pl.ds(v * 16, 16)] = src_g[j, 0, pl.ds(v * 16, 16)]
                    dst_c[pl.ds(v * 16, 16)] = dst_g[j, 0, pl.ds(v * 16, 16)]
                cp_a = pltpu.async_copy(h1_hbm.at[src_c], rows_a, sem_a)
                cp_b = pltpu.async_copy(h2_hbm.at[src_c], rows_b, sem_b)
                for v in range(K // 16):
                    s16 = src_c[pl.ds(v * 16, 16)]
                    d16 = dst_c[pl.ds(v * 16, 16)]
                    a_s = plsc.load_gather(tsrc, [s16])
                    ad_d = plsc.load_gather(tdst, [d16])
                    as_d = plsc.load_gather(tsrc, [d16])
                    e = a_s + ad_d
                    e = jnp.where(e >= 0.0, e, 0.2 * e)
                    c = as_d + ad_d
                    c = jnp.where(c >= 0.0, c, 0.2 * c)
                    w = jnp.exp(e - c)
                    wbuf[pl.ds(v * 16, 16)] = w
                    plsc.addupdate_scatter(dvec, [zi, d16], w)
                cp_a.wait()

                def srow_a(r, _):
                    wr = plsc.load_gather(wbuf, [jnp.full((16,), r, jnp.int32)])
                    for cc in range(D // 32):
                        rows_a[r, pl.ds(cc * 16, 16)] = rows_a[r, pl.ds(cc * 16, 16)] * wr
                    return 0

                lax.fori_loop(0, K, srow_a, 0)
                pltpu.sync_copy(rows_a, numer1_s.at[dst_c], add=True)
                cp_b.wait()

                def srow_b(r, _):
                    wr = plsc.load_gather(wbuf, [jnp.full((16,), r, jnp.int32)])
                    for cc in range(D // 32):
                        rows_b[r, pl.ds(cc * 16, 16)] = rows_b[r, pl.ds(cc * 16, 16)] * wr
                    return 0

                lax.fori_loop(0, K, srow_b, 0)
                pltpu.sync_copy(rows_b, numer2_s.at[dst_c], add=True)
                return 0

            lax.fori_loop(0, GRP, chunk, 0)
            return 0

        lax.fori_loop(0, NG, group, 0)
        plsc.subcore_barrier()

        pltpu.sync_copy(numer1_s.at[pl.ds(base, RPT)],
                        numer1_hbm.at[cid, pl.ds(base, RPT)])
        pltpu.sync_copy(numer2_s.at[pl.ds(base, RPT)],
                        numer2_hbm.at[cid, pl.ds(base, RPT)])
        pltpu.sync_copy(dvec, denom_hbm.at[wid])

    return edge_kernel


def _phase2(numer1, numer2, denom_t, h, bias2, lnw2, lnb2):
    bn = 1000

    def body(n1_ref, n2_ref, d_ref, h_ref, b_ref, w_ref, bb_ref, o_ref):
        num = jnp.concatenate(
            [n1_ref[0] + n1_ref[1], n2_ref[0] + n2_ref[1]], axis=-1) + h_ref[...]
        den = jnp.sum(d_ref[...], axis=-1, keepdims=True) + 1.0
        s = num / (den + 1e-16) + b_ref[...]
        mean = jnp.mean(s, axis=-1, keepdims=True)
        var = jnp.mean((s - mean) ** 2, axis=-1, keepdims=True)
        o_ref[...] = (s - mean) / jnp.sqrt(var + 1e-5) * w_ref[...] + bb_ref[...]

    return pl.pallas_call(
        body,
        grid=(N // bn,),
        in_specs=[pl.BlockSpec((NC, bn, D // 2), lambda i: (0, i, 0)),
                  pl.BlockSpec((NC, bn, D // 2), lambda i: (0, i, 0)),
                  pl.BlockSpec((bn, NW), lambda i: (i, 0)),
                  pl.BlockSpec((bn, D), lambda i: (i, 0)),
                  pl.BlockSpec((1, D), lambda i: (0, 0)),
                  pl.BlockSpec((1, D), lambda i: (0, 0)),
                  pl.BlockSpec((1, D), lambda i: (0, 0))],
        out_specs=pl.BlockSpec((bn, D), lambda i: (i, 0)),
        out_shape=jax.ShapeDtypeStruct((N, D), jnp.float32),
    )(numer1, numer2, denom_t, h, bias2, lnw2, lnb2)


def kernel(x, edge_index, W, att_src, att_dst, bias, ln_w, ln_b):
    ei = edge_index.astype(jnp.int32)
    src = ei[0].reshape(NW, NG, GRP, 1, K)
    dst = ei[1].reshape(NW, NG, GRP, 1, K)
    w_src = W @ att_src
    w_dst = W @ att_dst
    A8 = jnp.concatenate(
        [w_src[None], w_dst[None], jnp.zeros((6, D), jnp.float32)], axis=0)
    h, scal3 = _phase0(x, W, A8)
    tsrc = scal3[:, 0, :].reshape(N)
    tdst = scal3[:, 1, :].reshape(N)
    h1 = h[:, :D // 2]
    h2 = h[:, D // 2:]
    numer1, numer2, denom = _make_edge_kernel()(src, dst, tsrc, tdst, h1, h2)
    denom_t = denom.reshape(NW, N).T
    return _phase2(numer1, numer2, denom_t, h,
                   bias[None, :], ln_w[None, :], ln_b[None, :])


# speedup vs baseline: 1.1796x; 1.1796x over previous
import functools

import jax
import jax.numpy as jnp
from jax import lax
from jax.experimental import pallas as pl
from jax.experimental.pallas import tpu as pltpu
from jax.experimental.pallas import tpu_sc as plsc

N = 10000
E = 320000
D = 128
NC = 2
NS = 16
NW = NC * NS
EPW = E // NW
K = 80
NCH = EPW // K
GRP = 5
NG = NCH // GRP
NP = 10240
RPT = NP // NS


def _phase0(x, Wm, A8):
    bn = 1000

    def body(x_ref, w_ref, a_ref, h_ref, s_ref):
        xb = x_ref[...]
        h_ref[...] = jnp.dot(xb, w_ref[...], preferred_element_type=jnp.float32)
        m = lax.dot_general(a_ref[...], xb, (((1,), (1,)), ((), ())),
                            preferred_element_type=jnp.float32)
        s_ref[...] = m[None]

    return pl.pallas_call(
        body,
        grid=(N // bn,),
        in_specs=[pl.BlockSpec((bn, D), lambda i: (i, 0)),
                  pl.BlockSpec((D, D), lambda i: (0, 0)),
                  pl.BlockSpec((8, D), lambda i: (0, 0))],
        out_specs=[pl.BlockSpec((bn, D), lambda i: (i, 0)),
                   pl.BlockSpec((1, 8, bn), lambda i: (i, 0, 0))],
        out_shape=[jax.ShapeDtypeStruct((N, D), jnp.float32),
                   jax.ShapeDtypeStruct((N // bn, 8, bn), jnp.float32)],
    )(x, Wm, A8)


def _make_edge_kernel():
    mesh = plsc.VectorSubcoreMesh(core_axis_name="c", subcore_axis_name="s")

    @functools.partial(
        pl.kernel,
        mesh=mesh,
        compiler_params=pltpu.CompilerParams(needs_layout_passes=False),
        out_type=[jax.ShapeDtypeStruct((NC, NP, D), jnp.float32),
                  jax.ShapeDtypeStruct((NW, 1, N), jnp.float32)],
        scratch_types=[
            pltpu.VMEM((GRP, 1, K), jnp.int32),
            pltpu.VMEM((GRP, 1, K), jnp.int32),
            pltpu.VMEM((K,), jnp.int32),
            pltpu.VMEM((K,), jnp.int32),
            pltpu.VMEM((N,), jnp.float32),
            pltpu.VMEM((N,), jnp.float32),
            pltpu.VMEM((K,), jnp.float32),
            pltpu.VMEM((K, D), jnp.float32),
            pltpu.VMEM((1, N), jnp.float32),
            pltpu.VMEM_SHARED((NP, D), jnp.float32),
            pltpu.SemaphoreType.DMA,
        ])
    def edge_kernel(src_hbm, dst_hbm, tsrc_hbm, tdst_hbm, h_hbm,
                    numer_hbm, denom_hbm,
                    src_g, dst_g, src_c, dst_c, tsrc, tdst, wbuf, rows,
                    dvec, numer_s, sem):
        cid = lax.axis_index("c")
        sid = lax.axis_index("s")
        wid = sid * NC + cid
        base = sid * RPT

        pltpu.sync_copy(tsrc_hbm, tsrc)
        pltpu.sync_copy(tdst_hbm, tdst)

        zv = jnp.zeros((16,), jnp.float32)

        def zdv(i, _):
            dvec[0, pl.ds(i * 16, 16)] = zv
            return 0

        lax.fori_loop(0, N // 16, zdv, 0)

        def zrow(r, _):
            for cc in range(D // 16):
                rows[r, pl.ds(cc * 16, 16)] = zv
            return 0

        lax.fori_loop(0, K, zrow, 0)

        def zcopy(j, _):
            pltpu.sync_copy(rows, numer_s.at[pl.ds(base + j * K, K)])
            return 0

        lax.fori_loop(0, RPT // K, zcopy, 0)
        plsc.subcore_barrier()

        zi = jnp.zeros((16,), jnp.int32)

        def group(g, _):
            pltpu.sync_copy(src_hbm.at[wid, g], src_g)
            pltpu.sync_copy(dst_hbm.at[wid, g], dst_g)

            def chunk(j, _):
                for v in range(K // 16):
                    src_c[pl.ds(v * 16, 16)] = src_g[j, 0, pl.ds(v * 16, 16)]
                    dst_c[pl.ds(v * 16, 16)] = dst_g[j, 0, pl.ds(v * 16, 16)]
                cp = pltpu.async_copy(h_hbm.at[src_c], rows, sem)
                for v in range(K // 16):
                    s16 = src_c[pl.ds(v * 16, 16)]
                    d16 = dst_c[pl.ds(v * 16, 16)]
                    a_s = plsc.load_gather(tsrc, [s16])
                    ad_d = plsc.load_gather(tdst, [d16])
                    as_d = plsc.load_gather(tsrc, [d16])
                    e = a_s + ad_d
                    e = jnp.where(e >= 0.0, e, 0.2 * e)
                    c = as_d + ad_d
                    c = jnp.where(c >= 0.0, c, 0.2 * c)
                    w = jnp.exp(e - c)
                    wbuf[pl.ds(v * 16, 16)] = w
                    plsc.addupdate_scatter(dvec, [zi, d16], w)
                cp.wait()

                def srow(q, _):
                    for u in range(4):
                        r = q * 4 + u
                        wr = plsc.load_gather(wbuf, [jnp.full((16,), r, jnp.int32)])
                        for cc in range(D // 16):
                            rows[r, pl.ds(cc * 16, 16)] = rows[r, pl.ds(cc * 16, 16)] * wr
                    return 0

                lax.fori_loop(0, K // 4, srow, 0)
                pltpu.sync_copy(rows, numer_s.at[dst_c], add=True)
                return 0

            lax.fori_loop(0, GRP, chunk, 0)
            return 0

        lax.fori_loop(0, NG, group, 0)
        plsc.subcore_barrier()

        pltpu.sync_copy(numer_s.at[pl.ds(base, RPT)],
                        numer_hbm.at[cid, pl.ds(base, RPT)])
        pltpu.sync_copy(dvec, denom_hbm.at[wid])

    return edge_kernel


def _phase2(numer, denom_t, h, bias2, lnw2, lnb2):
    bn = 1000

    def body(n_ref, d_ref, h_ref, b_ref, w_ref, bb_ref, o_ref):
        num = n_ref[0] + n_ref[1] + h_ref[...]
        den = jnp.sum(d_ref[...], axis=-1, keepdims=True) + 1.0
        s = num / (den + 1e-16) + b_ref[...]
        mean = jnp.mean(s, axis=-1, keepdims=True)
        var = jnp.mean((s - mean) ** 2, axis=-1, keepdims=True)
        o_ref[...] = (s - mean) / jnp.sqrt(var + 1e-5) * w_ref[...] + bb_ref[...]

    return pl.pallas_call(
        body,
        grid=(N // bn,),
        in_specs=[pl.BlockSpec((NC, bn, D), lambda i: (0, i, 0)),
                  pl.BlockSpec((bn, NW), lambda i: (i, 0)),
                  pl.BlockSpec((bn, D), lambda i: (i, 0)),
                  pl.BlockSpec((1, D), lambda i: (0, 0)),
                  pl.BlockSpec((1, D), lambda i: (0, 0)),
                  pl.BlockSpec((1, D), lambda i: (0, 0))],
        out_specs=pl.BlockSpec((bn, D), lambda i: (i, 0)),
        out_shape=jax.ShapeDtypeStruct((N, D), jnp.float32),
    )(numer, denom_t, h, bias2, lnw2, lnb2)


def kernel(x, edge_index, W, att_src, att_dst, bias, ln_w, ln_b):
    ei = edge_index.astype(jnp.int32)
    src = ei[0].reshape(NW, NG, GRP, 1, K)
    dst = ei[1].reshape(NW, NG, GRP, 1, K)
    w_src = W @ att_src
    w_dst = W @ att_dst
    A8 = jnp.concatenate(
        [w_src[None], w_dst[None], jnp.zeros((6, D), jnp.float32)], axis=0)
    h, scal3 = _phase0(x, W, A8)
    tsrc = scal3[:, 0, :].reshape(N)
    tdst = scal3[:, 1, :].reshape(N)
    numer, denom = _make_edge_kernel()(src, dst, tsrc, tdst, h)
    denom_t = denom.reshape(NW, N).T
    return _phase2(numer, denom_t, h, bias[None, :], ln_w[None, :], ln_b[None, :])


# speedup vs baseline: 1.1854x; 1.0049x over previous
import functools

import jax
import jax.numpy as jnp
from jax import lax
from jax.experimental import pallas as pl
from jax.experimental.pallas import tpu as pltpu
from jax.experimental.pallas import tpu_sc as plsc

N = 10000
E = 320000
D = 128
NC = 2
NS = 16
NW = NC * NS
EPW = E // NW
K = 80
NCH = EPW // K
GRP = 5
NG = NCH // GRP
NP = 10240
RPT = NP // NS


def _phase0(x, Wm, A8):
    bn = 1000

    def body(x_ref, w_ref, a_ref, h_ref, s_ref):
        xb = x_ref[...]
        h_ref[...] = jnp.dot(xb, w_ref[...], preferred_element_type=jnp.float32)
        m = lax.dot_general(a_ref[...], xb, (((1,), (1,)), ((), ())),
                            preferred_element_type=jnp.float32)
        s_ref[...] = m[None]

    return pl.pallas_call(
        body,
        grid=(N // bn,),
        in_specs=[pl.BlockSpec((bn, D), lambda i: (i, 0)),
                  pl.BlockSpec((D, D), lambda i: (0, 0)),
                  pl.BlockSpec((8, D), lambda i: (0, 0))],
        out_specs=[pl.BlockSpec((bn, D), lambda i: (i, 0)),
                   pl.BlockSpec((1, 8, bn), lambda i: (i, 0, 0))],
        out_shape=[jax.ShapeDtypeStruct((N, D), jnp.float32),
                   jax.ShapeDtypeStruct((N // bn, 8, bn), jnp.float32)],
    )(x, Wm, A8)


def _make_edge_kernel():
    mesh = plsc.VectorSubcoreMesh(core_axis_name="c", subcore_axis_name="s")

    @functools.partial(
        pl.kernel,
        mesh=mesh,
        compiler_params=pltpu.CompilerParams(needs_layout_passes=False),
        out_type=[jax.ShapeDtypeStruct((NC, NP, D), jnp.float32),
                  jax.ShapeDtypeStruct((NW, 1, N), jnp.float32)],
        scratch_types=[
            pltpu.VMEM((GRP, 1, 2 * K), jnp.int32),
            pltpu.VMEM((K,), jnp.int32),
            pltpu.VMEM((K,), jnp.int32),
            pltpu.VMEM((N,), jnp.float32),
            pltpu.VMEM((N,), jnp.float32),
            pltpu.VMEM((K,), jnp.float32),
            pltpu.VMEM((K, D), jnp.float32),
            pltpu.VMEM((1, N), jnp.float32),
            pltpu.VMEM_SHARED((NP, D), jnp.float32),
            pltpu.SemaphoreType.DMA,
        ])
    def edge_kernel(sd_hbm, tsrc_hbm, tdst_hbm, h_hbm,
                    numer_hbm, denom_hbm,
                    sd_g, src_c, dst_c, tsrc, tdst, wbuf, rows,
                    dvec, numer_s, sem):
        cid = lax.axis_index("c")
        sid = lax.axis_index("s")
        wid = sid * NC + cid
        base = sid * RPT

        pltpu.sync_copy(tsrc_hbm, tsrc)
        pltpu.sync_copy(tdst_hbm, tdst)

        zv = jnp.zeros((16,), jnp.float32)

        def zdv(i, _):
            dvec[0, pl.ds(i * 16, 16)] = zv
            return 0

        lax.fori_loop(0, N // 16, zdv, 0)

        def zrow(r, _):
            for cc in range(D // 16):
                rows[r, pl.ds(cc * 16, 16)] = zv
            return 0

        lax.fori_loop(0, K, zrow, 0)

        def zcopy(j, _):
            pltpu.sync_copy(rows, numer_s.at[pl.ds(base + j * K, K)])
            return 0

        lax.fori_loop(0, RPT // K, zcopy, 0)
        plsc.subcore_barrier()

        zi = jnp.zeros((16,), jnp.int32)

        def group(g, _):
            pltpu.sync_copy(sd_hbm.at[wid, g], sd_g)

            def chunk(j, _):
                for v in range(K // 16):
                    src_c[pl.ds(v * 16, 16)] = sd_g[j, 0, pl.ds(v * 16, 16)]
                    dst_c[pl.ds(v * 16, 16)] = sd_g[j, 0, pl.ds(K + v * 16, 16)]
                cp = pltpu.async_copy(h_hbm.at[src_c], rows, sem)
                for v in range(K // 16):
                    s16 = src_c[pl.ds(v * 16, 16)]
                    d16 = dst_c[pl.ds(v * 16, 16)]
                    a_s = plsc.load_gather(tsrc, [s16])
                    ad_d = plsc.load_gather(tdst, [d16])
                    as_d = plsc.load_gather(tsrc, [d16])
                    e = a_s + ad_d
                    e = jnp.where(e >= 0.0, e, 0.2 * e)
                    c = as_d + ad_d
                    c = jnp.where(c >= 0.0, c, 0.2 * c)
                    w = jnp.exp(e - c)
                    wbuf[pl.ds(v * 16, 16)] = w
                    plsc.addupdate_scatter(dvec, [zi, d16], w)
                cp.wait()

                def srow(q, _):
                    for u in range(4):
                        r = q * 4 + u
                        wr = plsc.load_gather(wbuf, [jnp.full((16,), r, jnp.int32)])
                        for cc in range(D // 16):
                            rows[r, pl.ds(cc * 16, 16)] = rows[r, pl.ds(cc * 16, 16)] * wr
                    return 0

                lax.fori_loop(0, K // 4, srow, 0)
                pltpu.sync_copy(rows, numer_s.at[dst_c], add=True)
                return 0

            lax.fori_loop(0, GRP, chunk, 0)
            return 0

        lax.fori_loop(0, NG, group, 0)
        plsc.subcore_barrier()

        pltpu.sync_copy(numer_s.at[pl.ds(base, RPT)],
                        numer_hbm.at[cid, pl.ds(base, RPT)])
        pltpu.sync_copy(dvec, denom_hbm.at[wid])

    return edge_kernel


def _phase2(numer, denom_t, h, bias2, lnw2, lnb2):
    bn = 1000

    def body(n_ref, d_ref, h_ref, b_ref, w_ref, bb_ref, o_ref):
        num = n_ref[0] + n_ref[1] + h_ref[...]
        den = jnp.sum(d_ref[...], axis=-1, keepdims=True) + 1.0
        s = num / (den + 1e-16) + b_ref[...]
        mean = jnp.mean(s, axis=-1, keepdims=True)
        var = jnp.mean((s - mean) ** 2, axis=-1, keepdims=True)
        o_ref[...] = (s - mean) / jnp.sqrt(var + 1e-5) * w_ref[...] + bb_ref[...]

    return pl.pallas_call(
        body,
        grid=(N // bn,),
        in_specs=[pl.BlockSpec((NC, bn, D), lambda i: (0, i, 0)),
                  pl.BlockSpec((bn, NW), lambda i: (i, 0)),
                  pl.BlockSpec((bn, D), lambda i: (i, 0)),
                  pl.BlockSpec((1, D), lambda i: (0, 0)),
                  pl.BlockSpec((1, D), lambda i: (0, 0)),
                  pl.BlockSpec((1, D), lambda i: (0, 0))],
        out_specs=pl.BlockSpec((bn, D), lambda i: (i, 0)),
        out_shape=jax.ShapeDtypeStruct((N, D), jnp.float32),
    )(numer, denom_t, h, bias2, lnw2, lnb2)


def kernel(x, edge_index, W, att_src, att_dst, bias, ln_w, ln_b):
    ei = edge_index.astype(jnp.int32)
    sd = jnp.concatenate(
        [ei[0].reshape(NW, NG, GRP, 1, K), ei[1].reshape(NW, NG, GRP, 1, K)],
        axis=-1)
    w_src = W @ att_src
    w_dst = W @ att_dst
    A8 = jnp.concatenate(
        [w_src[None], w_dst[None], jnp.zeros((6, D), jnp.float32)], axis=0)
    h, scal3 = _phase0(x, W, A8)
    tsrc = scal3[:, 0, :].reshape(N)
    tdst = scal3[:, 1, :].reshape(N)
    numer, denom = _make_edge_kernel()(sd, tsrc, tdst, h)
    denom_t = denom.reshape(NW, N).T
    return _phase2(numer, denom_t, h, bias[None, :], ln_w[None, :], ln_b[None, :])
